# baseline (speedup 1.0000x reference)
import functools

import jax
import jax.numpy as jnp
from jax.experimental import pallas as pl
from jax.experimental.pallas import tpu as pltpu
from jax.experimental.pallas import tpu_sc as plsc

B, N, NUM_CLASS = 4, 20000, 3
PRE, POST, THRESH = 4096, 512, 0.7
BLK = 512
NBLK = PRE // BLK
W = 20480
IMIN = -2147483648


def _keys_body(cls_ref, keys_ref, labels_ref, tau_ref, ngt_ref):
    a = cls_ref[0]
    s0 = a[0:1]
    s1 = a[1:2]
    s2 = a[2:3]
    sc = jnp.maximum(jnp.maximum(s0, s1), s2)
    lab = jnp.where(s0 >= s1,
                    jnp.where(s0 >= s2, 0, 2),
                    jnp.where(s1 >= s2, 1, 2)).astype(jnp.int32)
    bits = jax.lax.bitcast_convert_type(sc, jnp.int32)
    mag = bits & jnp.int32(0x7FFFFFFF)
    key = jnp.where(bits < 0, -mag, mag)
    lane = jax.lax.broadcasted_iota(jnp.int32, (1, W), 1)
    key = jnp.where(lane < N, key, IMIN)
    keys_ref[0] = key
    labels_ref[0] = lab
    cnt0 = jnp.sum(jnp.where(key >= 0, 1, 0))
    t = jnp.where(cnt0 >= PRE, 0, IMIN)
    for bit in range(30, -1, -1):
        cand = t + jnp.int32(1 << bit)
        cnt = jnp.sum(jnp.where(key >= cand, 1, 0))
        t = jnp.where(cnt >= PRE, cand, t)
    tau_ref[0] = jnp.broadcast_to(t, (1, 1))
    ngt_ref[0] = jnp.broadcast_to(jnp.sum(jnp.where(key > t, 1, 0)), (1, 1))


def _keys(cls_t):
    return pl.pallas_call(
        _keys_body,
        grid=(B,),
        in_specs=[pl.BlockSpec((1, 3, W), lambda b: (b, 0, 0))],
        out_specs=[
            pl.BlockSpec((1, 1, W), lambda b: (b, 0, 0)),
            pl.BlockSpec((1, 1, W), lambda b: (b, 0, 0)),
            pl.BlockSpec((1, 1, 1), lambda b: (b, 0, 0)),
            pl.BlockSpec((1, 1, 1), lambda b: (b, 0, 0)),
        ],
        out_shape=[
            jax.ShapeDtypeStruct((B, 1, W), jnp.int32),
            jax.ShapeDtypeStruct((B, 1, W), jnp.int32),
            jax.ShapeDtypeStruct((B, 1, 1), jnp.int32),
            jax.ShapeDtypeStruct((B, 1, 1), jnp.int32),
        ],
    )(cls_t)


_SC_MESH = plsc.VectorSubcoreMesh(core_axis_name="c", subcore_axis_name="s")
_SC_PARAMS = pltpu.CompilerParams(needs_layout_passes=False)
NLANE = 16
NCORE = 2


def _sc_wid():
    return jax.lax.axis_index("s") * NCORE + jax.lax.axis_index("c")


def _compact_body(keys_hbm, tau_hbm, ngt_hbm, selidx_hbm, selkey_hbm,
                  keys_v, tau_v, ngt_v, si_v, sk_v):
    wid = _sc_wid()

    @pl.when(wid < B)
    def _():
        pltpu.sync_copy(keys_hbm.at[wid], keys_v)
        pltpu.sync_copy(tau_hbm.at[wid], tau_v)
        pltpu.sync_copy(ngt_hbm.at[wid], ngt_v)
        tau_s = jnp.max(tau_v[...])
        need_s = PRE - jnp.max(ngt_v[...])

        def step(i, carry):
            run_gt, run_eq = carry
            kv = keys_v[pl.ds(i * NLANE, NLANE)]
            gt = kv > tau_s
            eq = kv == tau_s
            gt32 = gt.astype(jnp.int32)
            eq32 = eq.astype(jnp.int32)
            gt_before = run_gt + jnp.cumsum(gt32) - gt32
            eq_before = run_eq + jnp.cumsum(eq32) - eq32
            sel = gt | (eq & (eq_before < need_s))
            pos = gt_before + jnp.minimum(eq_before, need_s)
            idxv = jax.lax.iota(jnp.int32, NLANE) + i * NLANE
            plsc.store_scatter(si_v, [pos], idxv, mask=sel)
            plsc.store_scatter(sk_v, [pos], kv, mask=sel)
            return (run_gt + jnp.sum(gt32), run_eq + jnp.sum(eq32))

        jax.lax.fori_loop(0, W // NLANE, step,
                          (jnp.int32(0), jnp.int32(0)))
        pltpu.sync_copy(si_v.at[pl.ds(0, PRE)], selidx_hbm.at[wid])
        pltpu.sync_copy(sk_v.at[pl.ds(0, PRE)], selkey_hbm.at[wid])


def _compact(keys, tau16, ngt16):
    return pl.kernel(
        _compact_body,
        out_type=[jax.ShapeDtypeStruct((B, PRE), jnp.int32),
                  jax.ShapeDtypeStruct((B, PRE), jnp.int32)],
        mesh=_SC_MESH,
        compiler_params=_SC_PARAMS,
        scratch_types=[pltpu.VMEM((W,), jnp.int32),
                       pltpu.VMEM((NLANE,), jnp.int32),
                       pltpu.VMEM((NLANE,), jnp.int32),
                       pltpu.VMEM((PRE + NLANE,), jnp.int32),
                       pltpu.VMEM((PRE + NLANE,), jnp.int32)],
    )(keys, tau16, ngt16)


def _permgather_body(rank_hbm, selidx_hbm, selkey_hbm, boxflat_hbm, labflat_hbm,
                     fields_hbm, skey_hbm, labs_hbm,
                     rank_v, sid_v, skv_v, sidx_v, skey_v,
                     idxl_v, if0, if1, if2, if3, if4, if5, if6,
                     df0, df1, df2, df3, df4, df5, df6, dsti_v, sem):
    idxf = [if0, if1, if2, if3, if4, if5, if6]
    dstf = [df0, df1, df2, df3, df4, df5, df6]
    wid = _sc_wid()

    @pl.when(wid < B)
    def _():
        pltpu.sync_copy(rank_hbm.at[wid], rank_v)
        pltpu.sync_copy(selidx_hbm.at[wid], sid_v)
        pltpu.sync_copy(selkey_hbm.at[wid], skv_v)

        UNR = 4

        def scat(i, c):
            for u in range(UNR):
                d = pl.ds((i * UNR + u) * NLANE, NLANE)
                rv = rank_v[d]
                plsc.store_scatter(sidx_v, [rv], sid_v[d])
                plsc.store_scatter(skey_v, [rv], skv_v[d])
            return c

        jax.lax.fori_loop(0, PRE // (NLANE * UNR), scat, jnp.int32(0))
        pltpu.sync_copy(skey_v, skey_hbm.at[wid])

        def mk(i, c):
            for u in range(UNR):
                d = pl.ds((i * UNR + u) * NLANE, NLANE)
                v = sidx_v[d]
                idxl_v[d] = v + wid * W
                b7 = (v + wid * N) * 7
                for fld in range(7):
                    idxf[fld][d] = b7 + fld
            return c

        jax.lax.fori_loop(0, PRE // (NLANE * UNR), mk, jnp.int32(0))

        cps = [pltpu.async_copy(labflat_hbm.at[idxl_v], dsti_v, sem)]
        for fld in range(7):
            cps.append(pltpu.async_copy(boxflat_hbm.at[idxf[fld]],
                                        dstf[fld], sem))
        for cp in cps:
            cp.wait()
        pltpu.sync_copy(dsti_v, labs_hbm.at[wid])
        for fld in range(7):
            pltpu.sync_copy(dstf[fld], fields_hbm.at[wid * 8 + fld])


def _permgather(rank, selidx, selkey, boxflat, labflat):
    return pl.kernel(
        _permgather_body,
        out_type=[jax.ShapeDtypeStruct((B * 8, PRE), jnp.float32),
                  jax.ShapeDtypeStruct((B, PRE), jnp.int32),
                  jax.ShapeDtypeStruct((B, PRE), jnp.int32)],
        mesh=_SC_MESH,
        compiler_params=_SC_PARAMS,
        scratch_types=([pltpu.VMEM((PRE,), jnp.int32)] * 6
                       + [pltpu.VMEM((PRE,), jnp.int32)] * 7
                       + [pltpu.VMEM((PRE,), jnp.float32)] * 7
                       + [pltpu.VMEM((PRE,), jnp.int32),
                          pltpu.SemaphoreType.DMA]),
    )(rank, selidx, selkey, boxflat, labflat)


def _rank_body(skey_ref, rank_ref):
    k = skey_ref[0]
    for b in range(NBLK):
        s = b * BLK
        kc = jax.lax.transpose(k[:, s:s + BLK], (1, 0))
        jj = jax.lax.broadcasted_iota(jnp.int32, (BLK, PRE), 1)
        ii = jax.lax.broadcasted_iota(jnp.int32, (BLK, PRE), 0) + s
        g = (k > kc) | ((k == kc) & (jj < ii))
        rank_ref[0, s:s + BLK] = jnp.sum(g.astype(jnp.int32), axis=1,
                                         keepdims=True)


def _rank(sel_key):
    return pl.pallas_call(
        _rank_body,
        grid=(B,),
        in_specs=[pl.BlockSpec((1, 1, PRE), lambda b: (b, 0, 0))],
        out_specs=[pl.BlockSpec((1, PRE, 1), lambda b: (b, 0, 0))],
        out_shape=[jax.ShapeDtypeStruct((B, PRE, 1), jnp.int32)],
    )(sel_key)[0]


def _nms_select_body(fields_ref, scores_ref, labels_ref, rois_ref, rsc_ref, rlb_ref,
                     valid_ref, kept_ref, cum_ref):
    f = fields_ref[0]
    skey = scores_ref[0]
    sc = jax.lax.bitcast_convert_type(
        jnp.where(skey < 0, (-skey) | IMIN, skey), jnp.float32)
    lb = labels_ref[0]

    cx = f[0:1]
    cy = f[1:2]
    dx = f[3:4]
    dy = f[4:5]
    x1 = cx - dx * 0.5
    x2 = cx + dx * 0.5
    y1 = cy - dy * 0.5
    y2 = cy + dy * 0.5
    areas = (x2 - x1) * (y2 - y1)

    valid_ref[...] = jnp.ones((1, PRE), jnp.float32)
    kept_ref[...] = jnp.zeros((1, PRE), jnp.float32)

    for b in range(NBLK):
        s = b * BLK
        nk = jnp.sum(kept_ref[...])

        @pl.when(nk < float(POST))
        def blk_body():
            fbT = jax.lax.transpose(f[:, s:s + BLK], (1, 0))
            x1c = fbT[:, 0:1] - fbT[:, 3:4] * 0.5
            x2c = fbT[:, 0:1] + fbT[:, 3:4] * 0.5
            y1c = fbT[:, 1:2] - fbT[:, 4:5] * 0.5
            y2c = fbT[:, 1:2] + fbT[:, 4:5] * 0.5
            areac = (x2c - x1c) * (y2c - y1c)

            x1b = x1[:, s:s + BLK]
            x2b = x2[:, s:s + BLK]
            y1b = y1[:, s:s + BLK]
            y2b = y2[:, s:s + BLK]
            areab = areas[:, s:s + BLK]
            vb = valid_ref[:, s:s + BLK]

            xx1 = jnp.maximum(x1c, x1b)
            yy1 = jnp.maximum(y1c, y1b)
            xx2 = jnp.minimum(x2c, x2b)
            yy2 = jnp.minimum(y2c, y2b)
            inter = jnp.clip(xx2 - xx1, 0.0) * jnp.clip(yy2 - yy1, 0.0)
            iou = inter / (areac + areab - inter + 1e-6)
            ii = jax.lax.broadcasted_iota(jnp.int32, (BLK, BLK), 0)
            jj = jax.lax.broadcasted_iota(jnp.int32, (BLK, BLK), 1)
            S = jnp.where((iou > THRESH) & (ii < jj), 1.0, 0.0)

            def fp_cond(c):
                return ~c[1]

            def fp_body(c):
                k, _ = c
                supp = jax.lax.dot_general(
                    k, S, (((1,), (0,)), ((), ())),
                    preferred_element_type=jnp.float32)
                k_new = vb * jnp.where(supp > 0.0, 0.0, 1.0)
                return (k_new, jnp.all(k_new == k))

            k0 = (vb, jnp.array(False))
            kb, _ = jax.lax.while_loop(fp_cond, fp_body, k0)

            kept_ref[:, s:s + BLK] = kb

            if b < NBLK - 1:
                kc = jax.lax.transpose(kb, (1, 0))
                for jc in range(b + 1, NBLK):
                    t = jc * BLK
                    xx1 = jnp.maximum(x1c, x1[:, t:t + BLK])
                    yy1 = jnp.maximum(y1c, y1[:, t:t + BLK])
                    xx2 = jnp.minimum(x2c, x2[:, t:t + BLK])
                    yy2 = jnp.minimum(y2c, y2[:, t:t + BLK])
                    inter = jnp.clip(xx2 - xx1, 0.0) * jnp.clip(yy2 - yy1, 0.0)
                    iou = inter / (areac + areas[:, t:t + BLK] - inter + 1e-6)
                    hit = jnp.where((iou > THRESH) & (kc > 0.0), 1.0, 0.0)
                    supp = jnp.max(hit, axis=0, keepdims=True)
                    valid_ref[:, t:t + BLK] = valid_ref[:, t:t + BLK] * (1.0 - supp)

    kept = kept_ref[...]
    li = jax.lax.broadcasted_iota(jnp.int32, (BLK, BLK), 0)
    lj = jax.lax.broadcasted_iota(jnp.int32, (BLK, BLK), 1)
    Lstrict = jnp.where(li < lj, 1.0, 0.0)
    carry = jnp.zeros((1, 1), jnp.float32)
    for c in range(NBLK):
        s = c * BLK
        kc = kept[:, s:s + BLK]
        cc = jax.lax.dot_general(
            kc, Lstrict, (((1,), (0,)), ((), ())),
            preferred_element_type=jnp.float32) + carry
        cum_ref[:, s:s + BLK] = cc
        carry = carry + jnp.sum(kc).reshape(1, 1)

    cum = cum_ref[...].astype(jnp.int32)
    slot = jax.lax.broadcasted_iota(jnp.int32, (POST, PRE), 0)
    OH = jnp.where((slot == cum) & (kept > 0.0), 1.0, 0.0)

    rois = jax.lax.dot_general(
        OH, f, (((1,), (1,)), ((), ())), preferred_element_type=jnp.float32)
    rsc = jax.lax.dot_general(
        OH, sc, (((1,), (1,)), ((), ())), preferred_element_type=jnp.float32)
    lbf = lb.astype(jnp.float32) + 1.0
    rlb = jax.lax.dot_general(
        OH, lbf, (((1,), (1,)), ((), ())), preferred_element_type=jnp.float32)

    rois_ref[0] = rois
    rsc_ref[0] = rsc
    rlb_ref[0] = rlb.astype(jnp.int32)


def _nms_select(fields, scores, labels):
    return pl.pallas_call(
        _nms_select_body,
        grid=(B,),
        in_specs=[
            pl.BlockSpec((1, 8, PRE), lambda b: (b, 0, 0)),
            pl.BlockSpec((1, 1, PRE), lambda b: (b, 0, 0)),
            pl.BlockSpec((1, 1, PRE), lambda b: (b, 0, 0)),
        ],
        out_specs=[
            pl.BlockSpec((1, POST, 8), lambda b: (b, 0, 0)),
            pl.BlockSpec((1, POST, 1), lambda b: (b, 0, 0)),
            pl.BlockSpec((1, POST, 1), lambda b: (b, 0, 0)),
        ],
        out_shape=[
            jax.ShapeDtypeStruct((B, POST, 8), jnp.float32),
            jax.ShapeDtypeStruct((B, POST, 1), jnp.float32),
            jax.ShapeDtypeStruct((B, POST, 1), jnp.int32),
        ],
        scratch_shapes=[
            pltpu.VMEM((1, PRE), jnp.float32),
            pltpu.VMEM((1, PRE), jnp.float32),
            pltpu.VMEM((1, PRE), jnp.float32),
        ],
    )(fields, scores, labels)


def kernel(batch_box_preds, batch_cls_preds):
    cls_t = jnp.pad(jnp.transpose(batch_cls_preds, (0, 2, 1)),
                    ((0, 0), (0, 0), (0, W - N)))
    keys3, labels3, tau3, ngt3 = _keys(cls_t)
    keys = keys3[:, 0, :]
    labels = labels3[:, 0, :]
    tau16 = jnp.broadcast_to(tau3[:, 0, :], (B, NLANE))
    ngt16 = jnp.broadcast_to(ngt3[:, 0, :], (B, NLANE))

    sel_idx, sel_key = _compact(keys, tau16, ngt16)

    rank = _rank(sel_key[:, None, :])[..., 0]

    boxflat = batch_box_preds.reshape(B * N * 7)
    labflat = labels.reshape(B * W)
    fields8, sorted_key, labels_sorted = _permgather(
        rank, sel_idx, sel_key, boxflat, labflat)

    fields = fields8.reshape(B, 8, PRE)
    rois8, rsc, rlb = _nms_select(fields, sorted_key[:, None, :],
                                  labels_sorted[:, None, :])
    return rois8[:, :, :7], rsc[..., 0], rlb[..., 0]


# speedup vs baseline: 1.1045x; 1.1045x over previous
import functools

import jax
import jax.numpy as jnp
from jax.experimental import pallas as pl
from jax.experimental.pallas import tpu as pltpu
from jax.experimental.pallas import tpu_sc as plsc

B, N, NUM_CLASS = 4, 20000, 3
PRE, POST, THRESH = 4096, 512, 0.7
BLK = 512
NBLK = PRE // BLK
W = 20480
IMIN = -2147483648


def _keys_body(cls_ref, keys_ref, labels_ref, tau_ref, ngt_ref):
    a = cls_ref[0]
    s0 = a[0:1]
    s1 = a[1:2]
    s2 = a[2:3]
    sc = jnp.maximum(jnp.maximum(s0, s1), s2)
    lab = jnp.where(s0 >= s1,
                    jnp.where(s0 >= s2, 0, 2),
                    jnp.where(s1 >= s2, 1, 2)).astype(jnp.int32)
    bits = jax.lax.bitcast_convert_type(sc, jnp.int32)
    mag = bits & jnp.int32(0x7FFFFFFF)
    key = jnp.where(bits < 0, -mag, mag)
    lane = jax.lax.broadcasted_iota(jnp.int32, (1, W), 1)
    key = jnp.where(lane < N, key, IMIN)
    keys_ref[0] = key
    labels_ref[0] = lab
    cnt0 = jnp.sum(jnp.where(key >= 0, 1, 0))
    t = jnp.where(cnt0 >= PRE, 0, IMIN)
    for bit in range(30, -1, -1):
        cand = t + jnp.int32(1 << bit)
        cnt = jnp.sum(jnp.where(key >= cand, 1, 0))
        t = jnp.where(cnt >= PRE, cand, t)
    tau_ref[0] = jnp.broadcast_to(t, (1, 1))
    ngt_ref[0] = jnp.broadcast_to(jnp.sum(jnp.where(key > t, 1, 0)), (1, 1))


def _keys(cls_t):
    return pl.pallas_call(
        _keys_body,
        grid=(B,),
        in_specs=[pl.BlockSpec((1, 3, W), lambda b: (b, 0, 0))],
        out_specs=[
            pl.BlockSpec((1, 1, W), lambda b: (b, 0, 0)),
            pl.BlockSpec((1, 1, W), lambda b: (b, 0, 0)),
            pl.BlockSpec((1, 1, 1), lambda b: (b, 0, 0)),
            pl.BlockSpec((1, 1, 1), lambda b: (b, 0, 0)),
        ],
        out_shape=[
            jax.ShapeDtypeStruct((B, 1, W), jnp.int32),
            jax.ShapeDtypeStruct((B, 1, W), jnp.int32),
            jax.ShapeDtypeStruct((B, 1, 1), jnp.int32),
            jax.ShapeDtypeStruct((B, 1, 1), jnp.int32),
        ],
    )(cls_t)


_SC_MESH = plsc.VectorSubcoreMesh(core_axis_name="c", subcore_axis_name="s")
_SC_PARAMS = pltpu.CompilerParams(needs_layout_passes=False)
NLANE = 16
NCORE = 2


def _sc_wid():
    return jax.lax.axis_index("s") * NCORE + jax.lax.axis_index("c")


def _compact_body(keys_hbm, tau_hbm, ngt_hbm, selidx_hbm, selkey_hbm,
                  keys_v, tau_v, ngt_v, si_v, sk_v):
    wid = _sc_wid()

    @pl.when(wid < B)
    def _():
        pltpu.sync_copy(keys_hbm.at[wid], keys_v)
        pltpu.sync_copy(tau_hbm.at[wid], tau_v)
        pltpu.sync_copy(ngt_hbm.at[wid], ngt_v)
        tau_s = jnp.max(tau_v[...])
        need_s = PRE - jnp.max(ngt_v[...])

        def step(i, carry):
            run_gt, run_eq = carry
            kv = keys_v[pl.ds(i * NLANE, NLANE)]
            gt = kv > tau_s
            eq = kv == tau_s
            gt32 = gt.astype(jnp.int32)
            eq32 = eq.astype(jnp.int32)
            gt_before = run_gt + jnp.cumsum(gt32) - gt32
            eq_before = run_eq + jnp.cumsum(eq32) - eq32
            sel = gt | (eq & (eq_before < need_s))
            pos = gt_before + jnp.minimum(eq_before, need_s)
            idxv = jax.lax.iota(jnp.int32, NLANE) + i * NLANE
            plsc.store_scatter(si_v, [pos], idxv, mask=sel)
            plsc.store_scatter(sk_v, [pos], kv, mask=sel)
            return (run_gt + jnp.sum(gt32), run_eq + jnp.sum(eq32))

        jax.lax.fori_loop(0, W // NLANE, step,
                          (jnp.int32(0), jnp.int32(0)))
        pltpu.sync_copy(si_v.at[pl.ds(0, PRE)], selidx_hbm.at[wid])
        pltpu.sync_copy(sk_v.at[pl.ds(0, PRE)], selkey_hbm.at[wid])


def _compact(keys, tau16, ngt16):
    return pl.kernel(
        _compact_body,
        out_type=[jax.ShapeDtypeStruct((B, PRE), jnp.int32),
                  jax.ShapeDtypeStruct((B, PRE), jnp.int32)],
        mesh=_SC_MESH,
        compiler_params=_SC_PARAMS,
        scratch_types=[pltpu.VMEM((W,), jnp.int32),
                       pltpu.VMEM((NLANE,), jnp.int32),
                       pltpu.VMEM((NLANE,), jnp.int32),
                       pltpu.VMEM((PRE + NLANE,), jnp.int32),
                       pltpu.VMEM((PRE + NLANE,), jnp.int32)],
    )(keys, tau16, ngt16)


def _permgather_body(rank_hbm, selidx_hbm, selkey_hbm, boxflat_hbm, labflat_hbm,
                     fields_hbm, skey_hbm, labs_hbm,
                     rank_v, sid_v, skv_v, sidx_v, skey_v,
                     idxl_v, if0, if1, if2, if3, if4, if5, if6,
                     df0, df1, df2, df3, df4, df5, df6, dsti_v, sem):
    idxf = [if0, if1, if2, if3, if4, if5, if6]
    dstf = [df0, df1, df2, df3, df4, df5, df6]
    wid = _sc_wid()

    @pl.when(wid < B)
    def _():
        pltpu.sync_copy(rank_hbm.at[wid], rank_v)
        pltpu.sync_copy(selidx_hbm.at[wid], sid_v)
        pltpu.sync_copy(selkey_hbm.at[wid], skv_v)

        UNR = 4

        def scat(i, c):
            for u in range(UNR):
                d = pl.ds((i * UNR + u) * NLANE, NLANE)
                rv = rank_v[d]
                plsc.store_scatter(sidx_v, [rv], sid_v[d])
                plsc.store_scatter(skey_v, [rv], skv_v[d])
            return c

        jax.lax.fori_loop(0, PRE // (NLANE * UNR), scat, jnp.int32(0))
        pltpu.sync_copy(skey_v, skey_hbm.at[wid])

        def mk(i, c):
            for u in range(UNR):
                d = pl.ds((i * UNR + u) * NLANE, NLANE)
                v = sidx_v[d]
                idxl_v[d] = v + wid * W
                for fld in range(7):
                    idxf[fld][d] = v + (wid * N + fld * (B * N))
            return c

        jax.lax.fori_loop(0, PRE // (NLANE * UNR), mk, jnp.int32(0))

        cps = [pltpu.async_copy(labflat_hbm.at[idxl_v], dsti_v, sem)]
        for fld in range(7):
            cps.append(pltpu.async_copy(boxflat_hbm.at[idxf[fld]],
                                        dstf[fld], sem))
        for cp in cps:
            cp.wait()
        pltpu.sync_copy(dsti_v, labs_hbm.at[wid])
        for fld in range(7):
            pltpu.sync_copy(dstf[fld], fields_hbm.at[wid * 8 + fld])


def _permgather(rank, selidx, selkey, boxflat, labflat):
    return pl.kernel(
        _permgather_body,
        out_type=[jax.ShapeDtypeStruct((B * 8, PRE), jnp.float32),
                  jax.ShapeDtypeStruct((B, PRE), jnp.int32),
                  jax.ShapeDtypeStruct((B, PRE), jnp.int32)],
        mesh=_SC_MESH,
        compiler_params=_SC_PARAMS,
        scratch_types=([pltpu.VMEM((PRE,), jnp.int32)] * 6
                       + [pltpu.VMEM((PRE,), jnp.int32)] * 7
                       + [pltpu.VMEM((PRE,), jnp.float32)] * 7
                       + [pltpu.VMEM((PRE,), jnp.int32),
                          pltpu.SemaphoreType.DMA]),
    )(rank, selidx, selkey, boxflat, labflat)


def _rank_body(skey_ref, rank_ref):
    k = skey_ref[0]
    for b in range(NBLK):
        s = b * BLK
        kc = jax.lax.transpose(k[:, s:s + BLK], (1, 0))
        jj = jax.lax.broadcasted_iota(jnp.int32, (BLK, PRE), 1)
        ii = jax.lax.broadcasted_iota(jnp.int32, (BLK, PRE), 0) + s
        g = (k > kc) | ((k == kc) & (jj < ii))
        rank_ref[0, s:s + BLK] = jnp.sum(g.astype(jnp.int32), axis=1,
                                         keepdims=True)


def _rank(sel_key):
    return pl.pallas_call(
        _rank_body,
        grid=(B,),
        in_specs=[pl.BlockSpec((1, 1, PRE), lambda b: (b, 0, 0))],
        out_specs=[pl.BlockSpec((1, PRE, 1), lambda b: (b, 0, 0))],
        out_shape=[jax.ShapeDtypeStruct((B, PRE, 1), jnp.int32)],
    )(sel_key)[0]


def _nms_select_body(fields_ref, scores_ref, labels_ref, rois_ref, rsc_ref, rlb_ref,
                     valid_ref, kept_ref, cum_ref):
    f = fields_ref[0]
    skey = scores_ref[0]
    sc = jax.lax.bitcast_convert_type(
        jnp.where(skey < 0, (-skey) | IMIN, skey), jnp.float32)
    lb = labels_ref[0]

    cx = f[0:1]
    cy = f[1:2]
    dx = f[3:4]
    dy = f[4:5]
    x1 = cx - dx * 0.5
    x2 = cx + dx * 0.5
    y1 = cy - dy * 0.5
    y2 = cy + dy * 0.5
    areas = (x2 - x1) * (y2 - y1)

    valid_ref[...] = jnp.ones((1, PRE), jnp.float32)
    kept_ref[...] = jnp.zeros((1, PRE), jnp.float32)

    for b in range(NBLK):
        s = b * BLK
        nk = jnp.sum(kept_ref[...])

        @pl.when(nk < float(POST))
        def blk_body():
            fbT = jax.lax.transpose(f[:, s:s + BLK], (1, 0))
            x1c = fbT[:, 0:1] - fbT[:, 3:4] * 0.5
            x2c = fbT[:, 0:1] + fbT[:, 3:4] * 0.5
            y1c = fbT[:, 1:2] - fbT[:, 4:5] * 0.5
            y2c = fbT[:, 1:2] + fbT[:, 4:5] * 0.5
            areac = (x2c - x1c) * (y2c - y1c)

            x1b = x1[:, s:s + BLK]
            x2b = x2[:, s:s + BLK]
            y1b = y1[:, s:s + BLK]
            y2b = y2[:, s:s + BLK]
            areab = areas[:, s:s + BLK]
            vb = valid_ref[:, s:s + BLK]

            xx1 = jnp.maximum(x1c, x1b)
            yy1 = jnp.maximum(y1c, y1b)
            xx2 = jnp.minimum(x2c, x2b)
            yy2 = jnp.minimum(y2c, y2b)
            inter = jnp.clip(xx2 - xx1, 0.0) * jnp.clip(yy2 - yy1, 0.0)
            iou = inter / (areac + areab - inter + 1e-6)
            ii = jax.lax.broadcasted_iota(jnp.int32, (BLK, BLK), 0)
            jj = jax.lax.broadcasted_iota(jnp.int32, (BLK, BLK), 1)
            S = jnp.where((iou > THRESH) & (ii < jj), 1.0, 0.0)

            def fp_cond(c):
                return ~c[1]

            def fp_body(c):
                k, _ = c
                supp = jax.lax.dot_general(
                    k, S, (((1,), (0,)), ((), ())),
                    preferred_element_type=jnp.float32)
                k_new = vb * jnp.where(supp > 0.0, 0.0, 1.0)
                return (k_new, jnp.all(k_new == k))

            k0 = (vb, jnp.array(False))
            kb, _ = jax.lax.while_loop(fp_cond, fp_body, k0)

            kept_ref[:, s:s + BLK] = kb

            if b < NBLK - 1:
                kc = jax.lax.transpose(kb, (1, 0))
                for jc in range(b + 1, NBLK):
                    t = jc * BLK
                    xx1 = jnp.maximum(x1c, x1[:, t:t + BLK])
                    yy1 = jnp.maximum(y1c, y1[:, t:t + BLK])
                    xx2 = jnp.minimum(x2c, x2[:, t:t + BLK])
                    yy2 = jnp.minimum(y2c, y2[:, t:t + BLK])
                    inter = jnp.clip(xx2 - xx1, 0.0) * jnp.clip(yy2 - yy1, 0.0)
                    iou = inter / (areac + areas[:, t:t + BLK] - inter + 1e-6)
                    hit = jnp.where((iou > THRESH) & (kc > 0.0), 1.0, 0.0)
                    supp = jnp.max(hit, axis=0, keepdims=True)
                    valid_ref[:, t:t + BLK] = valid_ref[:, t:t + BLK] * (1.0 - supp)

    kept = kept_ref[...]
    li = jax.lax.broadcasted_iota(jnp.int32, (BLK, BLK), 0)
    lj = jax.lax.broadcasted_iota(jnp.int32, (BLK, BLK), 1)
    Lstrict = jnp.where(li < lj, 1.0, 0.0)
    carry = jnp.zeros((1, 1), jnp.float32)
    for c in range(NBLK):
        s = c * BLK
        kc = kept[:, s:s + BLK]
        cc = jax.lax.dot_general(
            kc, Lstrict, (((1,), (0,)), ((), ())),
            preferred_element_type=jnp.float32) + carry
        cum_ref[:, s:s + BLK] = cc
        carry = carry + jnp.sum(kc).reshape(1, 1)

    cum = cum_ref[...].astype(jnp.int32)
    slot = jax.lax.broadcasted_iota(jnp.int32, (POST, PRE), 0)
    OH = jnp.where((slot == cum) & (kept > 0.0), 1.0, 0.0)

    rois = jax.lax.dot_general(
        OH, f, (((1,), (1,)), ((), ())), preferred_element_type=jnp.float32)
    rsc = jax.lax.dot_general(
        OH, sc, (((1,), (1,)), ((), ())), preferred_element_type=jnp.float32)
    lbf = lb.astype(jnp.float32) + 1.0
    rlb = jax.lax.dot_general(
        OH, lbf, (((1,), (1,)), ((), ())), preferred_element_type=jnp.float32)

    rois_ref[0] = rois
    rsc_ref[0] = rsc
    rlb_ref[0] = rlb.astype(jnp.int32)


def _nms_select(fields, scores, labels):
    return pl.pallas_call(
        _nms_select_body,
        grid=(B,),
        in_specs=[
            pl.BlockSpec((1, 8, PRE), lambda b: (b, 0, 0)),
            pl.BlockSpec((1, 1, PRE), lambda b: (b, 0, 0)),
            pl.BlockSpec((1, 1, PRE), lambda b: (b, 0, 0)),
        ],
        out_specs=[
            pl.BlockSpec((1, POST, 8), lambda b: (b, 0, 0)),
            pl.BlockSpec((1, POST, 1), lambda b: (b, 0, 0)),
            pl.BlockSpec((1, POST, 1), lambda b: (b, 0, 0)),
        ],
        out_shape=[
            jax.ShapeDtypeStruct((B, POST, 8), jnp.float32),
            jax.ShapeDtypeStruct((B, POST, 1), jnp.float32),
            jax.ShapeDtypeStruct((B, POST, 1), jnp.int32),
        ],
        scratch_shapes=[
            pltpu.VMEM((1, PRE), jnp.float32),
            pltpu.VMEM((1, PRE), jnp.float32),
            pltpu.VMEM((1, PRE), jnp.float32),
        ],
    )(fields, scores, labels)


def kernel(batch_box_preds, batch_cls_preds):
    cls_t = jnp.pad(jnp.transpose(batch_cls_preds, (0, 2, 1)),
                    ((0, 0), (0, 0), (0, W - N)))
    keys3, labels3, tau3, ngt3 = _keys(cls_t)
    keys = keys3[:, 0, :]
    labels = labels3[:, 0, :]
    tau16 = jnp.broadcast_to(tau3[:, 0, :], (B, NLANE))
    ngt16 = jnp.broadcast_to(ngt3[:, 0, :], (B, NLANE))

    sel_idx, sel_key = _compact(keys, tau16, ngt16)

    rank = _rank(sel_key[:, None, :])[..., 0]

    boxflat = jnp.transpose(batch_box_preds, (2, 0, 1)).reshape(7 * B * N)
    labflat = labels.reshape(B * W)
    fields8, sorted_key, labels_sorted = _permgather(
        rank, sel_idx, sel_key, boxflat, labflat)

    fields = fields8.reshape(B, 8, PRE)
    rois8, rsc, rlb = _nms_select(fields, sorted_key[:, None, :],
                                  labels_sorted[:, None, :])
    return rois8[:, :, :7], rsc[..., 0], rlb[..., 0]
